# Initial kernel scaffold; baseline (speedup 1.0000x reference)
import functools

import jax
import jax.numpy as jnp
from jax import lax
from jax.experimental import pallas as pl
from jax.experimental.pallas import tpu as pltpu
from jax.experimental.pallas import tpu_sc as plsc

NC = 2
NS = 16
LANES = 16
CHUNK = 128


def _sc_mesh():
    return plsc.VectorSubcoreMesh(core_axis_name="c", subcore_axis_name="s")


def _sc_degree(dst, w, npad):
    ep = dst.shape[0]
    chunks_per_tile = ep // (NC * NS * CHUNK)
    rows_per_tile = npad // NS
    zeros = jnp.zeros((rows_per_tile, LANES), jnp.float32)

    @functools.partial(
        pl.kernel,
        out_type=jax.ShapeDtypeStruct((NC, npad, LANES), jnp.float32),
        mesh=_sc_mesh(),
        scratch_types=[
            pltpu.VMEM((CHUNK,), jnp.int32),
            pltpu.VMEM((CHUNK,), jnp.float32),
            pltpu.VMEM((CHUNK, LANES), jnp.float32),
            pltpu.VMEM_SHARED((npad, LANES), jnp.float32),
        ],
    )
    def k(dst_hbm, w_hbm, zero_hbm, out_hbm, idx_v, w_v, s_v, acc_sh):
        cid = lax.axis_index("c")
        sid = lax.axis_index("s")
        wid = cid * NS + sid
        my_rows = pl.ds(sid * rows_per_tile, rows_per_tile)
        pltpu.sync_copy(zero_hbm, acc_sh.at[my_rows])
        plsc.subcore_barrier()

        @pl.loop(0, chunks_per_tile)
        def _(ci):
            base = (wid * chunks_per_tile + ci) * CHUNK
            pltpu.sync_copy(dst_hbm.at[pl.ds(base, CHUNK)], idx_v)
            pltpu.sync_copy(w_hbm.at[pl.ds(base, CHUNK)], w_v)

            @pl.loop(0, CHUNK)
            def _(j):
                sp = jnp.full((LANES,), j, jnp.int32)
                s_v[j, :] = plsc.load_gather(w_v, [sp])

            pltpu.sync_copy(s_v, acc_sh.at[idx_v], add=True)

        plsc.subcore_barrier()
        pltpu.sync_copy(acc_sh.at[my_rows], out_hbm.at[cid].at[my_rows])

    return k(dst, w, zeros)


def _sc_aggregate(ht, src, dst, w, npad):
    ep = src.shape[0]
    d = ht.shape[1]
    chunks_per_tile = ep // (NC * NS * CHUNK)
    rows_per_tile = npad // NS
    zeros = jnp.zeros((rows_per_tile, d), jnp.float32)

    @functools.partial(
        pl.kernel,
        out_type=jax.ShapeDtypeStruct((NC, npad, d), jnp.float32),
        mesh=_sc_mesh(),
        scratch_types=[
            pltpu.VMEM((CHUNK,), jnp.int32),
            pltpu.VMEM((CHUNK,), jnp.int32),
            pltpu.VMEM((CHUNK,), jnp.float32),
            pltpu.VMEM((CHUNK, d), jnp.float32),
            pltpu.VMEM_SHARED((npad, d), jnp.float32),
            pltpu.SemaphoreType.DMA,
        ],
    )
    def k(ht_hbm, src_hbm, dst_hbm, w_hbm, zero_hbm, out_hbm,
          si_v, di_v, w_v, rows_v, acc_sh, sem):
        cid = lax.axis_index("c")
        sid = lax.axis_index("s")
        wid = cid * NS + sid
        my_rows = pl.ds(sid * rows_per_tile, rows_per_tile)
        pltpu.sync_copy(zero_hbm, acc_sh.at[my_rows])
        plsc.subcore_barrier()

        @pl.loop(0, chunks_per_tile)
        def _(ci):
            base = (wid * chunks_per_tile + ci) * CHUNK
            pltpu.sync_copy(src_hbm.at[pl.ds(base, CHUNK)], si_v)
            pltpu.sync_copy(dst_hbm.at[pl.ds(base, CHUNK)], di_v)
            pltpu.sync_copy(w_hbm.at[pl.ds(base, CHUNK)], w_v)
            pltpu.async_copy(ht_hbm.at[si_v], rows_v, sem).wait()

            @pl.loop(0, CHUNK)
            def _(j):
                sp = jnp.full((LANES,), j, jnp.int32)
                wj = plsc.load_gather(w_v, [sp])
                for kk in range(d // LANES):
                    sl = pl.ds(kk * LANES, LANES)
                    rows_v[j, sl] = rows_v[j, sl] * wj

            pltpu.sync_copy(rows_v, acc_sh.at[di_v], add=True)

        plsc.subcore_barrier()
        pltpu.sync_copy(acc_sh.at[my_rows], out_hbm.at[cid].at[my_rows])

    return k(ht, src, dst, w, zeros)


def _tc(body, out_shape, *args):
    return pl.pallas_call(body, out_shape=out_shape)(*args)


def _mm_body(x_ref, w_ref, o_ref):
    o_ref[...] = jnp.dot(x_ref[...], w_ref[...],
                         preferred_element_type=jnp.float32)


def _dinv_ht_body(p0_ref, p1_ref, xw_ref, dinv_ref, ht_ref):
    dinv = lax.rsqrt(1.0 + p0_ref[...] + p1_ref[...])
    dinv_ref[...] = dinv
    ht_ref[...] = dinv * xw_ref[...]


def _combine_body(dinv_ref, a0_ref, a1_ref, hlin_ref, w_ref, b_ref,
                  olin_ref, oht_ref):
    dinv = dinv_ref[...]
    h = jnp.maximum(
        dinv * (a0_ref[...] + a1_ref[...])
        + dinv * dinv * hlin_ref[...] + b_ref[...], 0.0)
    hw = jnp.dot(h, w_ref[...], preferred_element_type=jnp.float32)
    olin_ref[...] = hw
    oht_ref[...] = dinv * hw


def _final_body(dinv_ref, a0_ref, a1_ref, hlin_ref, wh_ref, b_ref, bh_ref,
                o_ref):
    dinv = dinv_ref[...]
    h = jnp.maximum(
        dinv * (a0_ref[...] + a1_ref[...])
        + dinv * dinv * hlin_ref[...] + b_ref[...], 0.0)
    o_ref[...] = jnp.dot(h, wh_ref[...],
                         preferred_element_type=jnp.float32) + bh_ref[...]


@jax.jit
def kernel(x, edge_index, edge_weight, W1, b1, W2, b2, Wh, bh):
    n = x.shape[0]
    e = edge_weight.shape[0]
    npad = ((n + NS * 8 - 1) // (NS * 8)) * (NS * 8)
    egrain = NC * NS * CHUNK
    ep = ((e + egrain - 1) // egrain) * egrain

    src = edge_index[0].astype(jnp.int32)
    dst = edge_index[1].astype(jnp.int32)
    w = edge_weight.astype(jnp.float32)
    if ep != e:
        pad = ep - e
        src = jnp.concatenate([src, jnp.zeros((pad,), jnp.int32)])
        dst = jnp.concatenate([dst, jnp.zeros((pad,), jnp.int32)])
        w = jnp.concatenate([w, jnp.zeros((pad,), jnp.float32)])

    f32 = jnp.float32
    dh = W1.shape[1]

    degp = _sc_degree(dst, w, npad)
    xw1 = _tc(_mm_body, jax.ShapeDtypeStruct((n, dh), f32), x, W1)

    p0 = degp[0, :n, 0:1]
    p1 = degp[1, :n, 0:1]
    dinv, h1t = _tc(
        _dinv_ht_body,
        (jax.ShapeDtypeStruct((n, 1), f32),
         jax.ShapeDtypeStruct((n, dh), f32)),
        p0, p1, xw1)

    agg1 = _sc_aggregate(h1t, src, dst, w, npad)
    h1w2, h2t = _tc(
        _combine_body,
        (jax.ShapeDtypeStruct((n, dh), f32),
         jax.ShapeDtypeStruct((n, dh), f32)),
        dinv, agg1[0, :n, :], agg1[1, :n, :], xw1, W2, b1.reshape(1, dh))

    agg2 = _sc_aggregate(h2t, src, dst, w, npad)
    scores = _tc(
        _final_body,
        jax.ShapeDtypeStruct((n, 1), f32),
        dinv, agg2[0, :n, :], agg2[1, :n, :], h1w2, Wh,
        b2.reshape(1, dh), bh.reshape(1, 1))

    return jnp.squeeze(scores, -1)


# speedup vs baseline: 9.5784x; 9.5784x over previous
import dataclasses
import functools

import jax
import jax.numpy as jnp
from jax import lax
from jax.experimental import pallas as pl
from jax.experimental.pallas import tpu as pltpu
from jax.experimental.pallas import tpu_sc as plsc

NC = 2
NS = 16
LANES = 16
CHUNK = 128


def _sc_mesh():
    return plsc.VectorSubcoreMesh(core_axis_name="c", subcore_axis_name="s")


def _sc_params():
    cp = pltpu.CompilerParams()
    if "needs_layout_passes" in pltpu.CompilerParams.__dataclass_fields__:
        cp = dataclasses.replace(cp, needs_layout_passes=False)
    cp = dataclasses.replace(cp, use_tc_tiling_on_sc=False)
    return cp


def _sc_degree(dst, w, npad):
    ep = dst.shape[0]
    chunks_per_tile = ep // (NC * NS * CHUNK)
    rows_per_tile = npad // NS
    zeros = jnp.zeros((rows_per_tile, LANES), jnp.float32)

    @functools.partial(
        pl.kernel,
        out_type=jax.ShapeDtypeStruct((NC, npad, LANES), jnp.float32),
        mesh=_sc_mesh(),
        scratch_types=[
            pltpu.VMEM((CHUNK,), jnp.int32),
            pltpu.VMEM((CHUNK,), jnp.float32),
            pltpu.VMEM((CHUNK, LANES), jnp.float32),
            pltpu.VMEM_SHARED((npad, LANES), jnp.float32),
        ],
        compiler_params=_sc_params(),
    )
    def k(dst_hbm, w_hbm, zero_hbm, out_hbm, idx_v, w_v, s_v, acc_sh):
        cid = lax.axis_index("c")
        sid = lax.axis_index("s")
        wid = cid * NS + sid
        my_rows = pl.ds(sid * rows_per_tile, rows_per_tile)
        pltpu.sync_copy(zero_hbm, acc_sh.at[my_rows])
        plsc.subcore_barrier()

        @pl.loop(0, chunks_per_tile)
        def _(ci):
            base = (wid * chunks_per_tile + ci) * CHUNK
            pltpu.sync_copy(dst_hbm.at[pl.ds(base, CHUNK)], idx_v)
            pltpu.sync_copy(w_hbm.at[pl.ds(base, CHUNK)], w_v)

            @pl.loop(0, CHUNK)
            def _(j):
                sp = jnp.full((LANES,), j, jnp.int32)
                s_v[j, :] = plsc.load_gather(w_v, [sp])

            pltpu.sync_copy(s_v, acc_sh.at[idx_v], add=True)

        plsc.subcore_barrier()
        pltpu.sync_copy(acc_sh.at[my_rows], out_hbm.at[cid].at[my_rows])

    return k(dst, w, zeros)


def _sc_aggregate(ht, src, dst, w, npad):
    ep = src.shape[0]
    d = ht.shape[1]
    chunks_per_tile = ep // (NC * NS * CHUNK)
    rows_per_tile = npad // NS
    zeros = jnp.zeros((rows_per_tile, d), jnp.float32)

    @functools.partial(
        pl.kernel,
        out_type=jax.ShapeDtypeStruct((NC, npad, d), jnp.float32),
        mesh=_sc_mesh(),
        scratch_types=[
            pltpu.VMEM((CHUNK,), jnp.int32),
            pltpu.VMEM((CHUNK,), jnp.int32),
            pltpu.VMEM((CHUNK,), jnp.float32),
            pltpu.VMEM((CHUNK, d), jnp.float32),
            pltpu.VMEM_SHARED((npad, d), jnp.float32),
            pltpu.SemaphoreType.DMA,
        ],
        compiler_params=_sc_params(),
    )
    def k(ht_hbm, src_hbm, dst_hbm, w_hbm, zero_hbm, out_hbm,
          si_v, di_v, w_v, rows_v, acc_sh, sem):
        cid = lax.axis_index("c")
        sid = lax.axis_index("s")
        wid = cid * NS + sid
        my_rows = pl.ds(sid * rows_per_tile, rows_per_tile)
        pltpu.sync_copy(zero_hbm, acc_sh.at[my_rows])
        plsc.subcore_barrier()

        @pl.loop(0, chunks_per_tile)
        def _(ci):
            base = (wid * chunks_per_tile + ci) * CHUNK
            pltpu.sync_copy(src_hbm.at[pl.ds(base, CHUNK)], si_v)
            pltpu.sync_copy(dst_hbm.at[pl.ds(base, CHUNK)], di_v)
            pltpu.sync_copy(w_hbm.at[pl.ds(base, CHUNK)], w_v)
            pltpu.async_copy(ht_hbm.at[si_v], rows_v, sem).wait()

            @pl.loop(0, CHUNK)
            def _(j):
                sp = jnp.full((LANES,), j, jnp.int32)
                wj = plsc.load_gather(w_v, [sp])
                for kk in range(d // LANES):
                    sl = pl.ds(kk * LANES, LANES)
                    rows_v[j, sl] = rows_v[j, sl] * wj

            pltpu.sync_copy(rows_v, acc_sh.at[di_v], add=True)

        plsc.subcore_barrier()
        pltpu.sync_copy(acc_sh.at[my_rows], out_hbm.at[cid].at[my_rows])

    return k(ht, src, dst, w, zeros)


def _tc(body, out_shape, *args):
    return pl.pallas_call(body, out_shape=out_shape)(*args)


def _mm_body(x_ref, w_ref, o_ref):
    o_ref[...] = jnp.dot(x_ref[...], w_ref[...],
                         preferred_element_type=jnp.float32)


def _dinv_ht_body(p0_ref, p1_ref, xw_ref, dinv_ref, ht_ref):
    dinv = lax.rsqrt(1.0 + p0_ref[...] + p1_ref[...])
    dinv_ref[...] = dinv
    ht_ref[...] = dinv * xw_ref[...]


def _combine_body(dinv_ref, a0_ref, a1_ref, hlin_ref, w_ref, b_ref,
                  olin_ref, oht_ref):
    dinv = dinv_ref[...]
    h = jnp.maximum(
        dinv * (a0_ref[...] + a1_ref[...])
        + dinv * dinv * hlin_ref[...] + b_ref[...], 0.0)
    hw = jnp.dot(h, w_ref[...], preferred_element_type=jnp.float32)
    olin_ref[...] = hw
    oht_ref[...] = dinv * hw


def _final_body(dinv_ref, a0_ref, a1_ref, hlin_ref, wh_ref, b_ref, bh_ref,
                o_ref):
    dinv = dinv_ref[...]
    h = jnp.maximum(
        dinv * (a0_ref[...] + a1_ref[...])
        + dinv * dinv * hlin_ref[...] + b_ref[...], 0.0)
    o_ref[...] = jnp.dot(h, wh_ref[...],
                         preferred_element_type=jnp.float32) + bh_ref[...]


@jax.jit
def kernel(x, edge_index, edge_weight, W1, b1, W2, b2, Wh, bh):
    n = x.shape[0]
    e = edge_weight.shape[0]
    npad = ((n + NS * 8 - 1) // (NS * 8)) * (NS * 8)
    egrain = NC * NS * CHUNK
    ep = ((e + egrain - 1) // egrain) * egrain

    src = edge_index[0].astype(jnp.int32)
    dst = edge_index[1].astype(jnp.int32)
    w = edge_weight.astype(jnp.float32)
    if ep != e:
        pad = ep - e
        src = jnp.concatenate([src, jnp.zeros((pad,), jnp.int32)])
        dst = jnp.concatenate([dst, jnp.zeros((pad,), jnp.int32)])
        w = jnp.concatenate([w, jnp.zeros((pad,), jnp.float32)])

    f32 = jnp.float32
    dh = W1.shape[1]

    degp = _sc_degree(dst, w, npad)
    xw1 = _tc(_mm_body, jax.ShapeDtypeStruct((n, dh), f32), x, W1)

    p0 = degp[0, :n, 0:1]
    p1 = degp[1, :n, 0:1]
    dinv, h1t = _tc(
        _dinv_ht_body,
        (jax.ShapeDtypeStruct((n, 1), f32),
         jax.ShapeDtypeStruct((n, dh), f32)),
        p0, p1, xw1)

    agg1 = _sc_aggregate(h1t, src, dst, w, npad)
    h1w2, h2t = _tc(
        _combine_body,
        (jax.ShapeDtypeStruct((n, dh), f32),
         jax.ShapeDtypeStruct((n, dh), f32)),
        dinv, agg1[0, :n, :], agg1[1, :n, :], xw1, W2, b1.reshape(1, dh))

    agg2 = _sc_aggregate(h2t, src, dst, w, npad)
    scores = _tc(
        _final_body,
        jax.ShapeDtypeStruct((n, 1), f32),
        dinv, agg2[0, :n, :], agg2[1, :n, :], h1w2, Wh,
        b2.reshape(1, dh), bh.reshape(1, 1))

    return jnp.squeeze(scores, -1)


# speedup vs baseline: 12.4602x; 1.3009x over previous
import dataclasses
import functools

import jax
import jax.numpy as jnp
from jax import lax
from jax.experimental import pallas as pl
from jax.experimental.pallas import tpu as pltpu
from jax.experimental.pallas import tpu_sc as plsc

NC = 2
NS = 16
LANES = 16
CHUNK = 128


def _sc_mesh():
    return plsc.VectorSubcoreMesh(core_axis_name="c", subcore_axis_name="s")


def _sc_params():
    cp = pltpu.CompilerParams()
    if "needs_layout_passes" in pltpu.CompilerParams.__dataclass_fields__:
        cp = dataclasses.replace(cp, needs_layout_passes=False)
    cp = dataclasses.replace(cp, use_tc_tiling_on_sc=False)
    return cp


def _sc_degree(dst, w, npad):
    ep = dst.shape[0]
    chunks_per_tile = ep // (NC * NS * CHUNK)
    rows_per_tile = npad // NS
    zeros = jnp.zeros((rows_per_tile, LANES), jnp.float32)

    @functools.partial(
        pl.kernel,
        out_type=jax.ShapeDtypeStruct((NC, npad, LANES), jnp.float32),
        mesh=_sc_mesh(),
        scratch_types=[
            pltpu.VMEM((CHUNK,), jnp.int32),
            pltpu.VMEM((CHUNK,), jnp.float32),
            pltpu.VMEM((CHUNK, LANES), jnp.float32),
            pltpu.VMEM_SHARED((npad, LANES), jnp.float32),
        ],
        compiler_params=_sc_params(),
    )
    def k(dst_hbm, w_hbm, zero_hbm, out_hbm, idx_v, w_v, s_v, acc_sh):
        cid = lax.axis_index("c")
        sid = lax.axis_index("s")
        wid = cid * NS + sid
        my_rows = pl.ds(sid * rows_per_tile, rows_per_tile)
        pltpu.sync_copy(zero_hbm, acc_sh.at[my_rows])
        plsc.subcore_barrier()

        @pl.loop(0, chunks_per_tile)
        def _(ci):
            base = (wid * chunks_per_tile + ci) * CHUNK
            pltpu.sync_copy(dst_hbm.at[pl.ds(base, CHUNK)], idx_v)
            pltpu.sync_copy(w_hbm.at[pl.ds(base, CHUNK)], w_v)

            @pl.loop(0, CHUNK)
            def _(j):
                sp = jnp.full((LANES,), j, jnp.int32)
                s_v[j, :] = plsc.load_gather(w_v, [sp])

            pltpu.sync_copy(s_v, acc_sh.at[idx_v], add=True)

        plsc.subcore_barrier()
        pltpu.sync_copy(acc_sh.at[my_rows], out_hbm.at[cid].at[my_rows])

    return k(dst, w, zeros)


def _sc_aggregate(ht, src, dst, w, npad):
    ep = src.shape[0]
    d = ht.shape[1]
    cpt = ep // (NC * NS * CHUNK)
    rows_per_tile = npad // NS
    zeros = jnp.zeros((rows_per_tile, d), jnp.float32)

    @functools.partial(
        pl.kernel,
        out_type=jax.ShapeDtypeStruct((NC, npad, d), jnp.float32),
        mesh=_sc_mesh(),
        scratch_types=[
            pltpu.VMEM((CHUNK,), jnp.int32),
            pltpu.VMEM((CHUNK,), jnp.int32),
            pltpu.VMEM((CHUNK,), jnp.int32),
            pltpu.VMEM((CHUNK,), jnp.int32),
            pltpu.VMEM((CHUNK,), jnp.float32),
            pltpu.VMEM((CHUNK,), jnp.float32),
            pltpu.VMEM((CHUNK, d), jnp.float32),
            pltpu.VMEM((CHUNK, d), jnp.float32),
            pltpu.VMEM_SHARED((npad, d), jnp.float32),
            pltpu.SemaphoreType.DMA,
            pltpu.SemaphoreType.DMA,
            pltpu.SemaphoreType.DMA,
            pltpu.SemaphoreType.DMA,
        ],
        compiler_params=_sc_params(),
    )
    def k(ht_hbm, src_hbm, dst_hbm, w_hbm, zero_hbm, out_hbm,
          si0, si1, di0, di1, w0, w1, r0, r1, acc_sh,
          isem0, isem1, gsem0, gsem1):
        cid = lax.axis_index("c")
        sid = lax.axis_index("s")
        wid = cid * NS + sid
        my_rows = pl.ds(sid * rows_per_tile, rows_per_tile)
        base0 = wid * cpt * CHUNK

        def start_idx(base, si, di, wb, sem):
            pltpu.async_copy(src_hbm.at[pl.ds(base, CHUNK)], si, sem)
            pltpu.async_copy(dst_hbm.at[pl.ds(base, CHUNK)], di, sem)
            pltpu.async_copy(w_hbm.at[pl.ds(base, CHUNK)], wb, sem)

        def wait_idx(si, di, wb, sem):
            pltpu.make_async_copy(src_hbm.at[pl.ds(0, CHUNK)], si, sem).wait()
            pltpu.make_async_copy(dst_hbm.at[pl.ds(0, CHUNK)], di, sem).wait()
            pltpu.make_async_copy(w_hbm.at[pl.ds(0, CHUNK)], wb, sem).wait()

        def scale_scatter(rows, wb, di):
            @pl.loop(0, CHUNK)
            def _(j):
                sp = jnp.full((LANES,), j, jnp.int32)
                wj = plsc.load_gather(wb, [sp])
                for kk in range(d // LANES):
                    sl = pl.ds(kk * LANES, LANES)
                    rows[j, sl] = rows[j, sl] * wj

            pltpu.sync_copy(rows, acc_sh.at[di], add=True)

        start_idx(base0, si0, di0, w0, isem0)
        pltpu.sync_copy(zero_hbm, acc_sh.at[my_rows])
        plsc.subcore_barrier()

        @pl.loop(0, cpt, step=2)
        def _(c):
            ba = base0 + c * CHUNK
            wait_idx(si0, di0, w0, isem0)
            gd0 = pltpu.async_copy(ht_hbm.at[si0], r0, gsem0)

            @pl.when(c > 0)
            def _():
                scale_scatter(r1, w1, di1)

            start_idx(ba + CHUNK, si1, di1, w1, isem1)
            gd0.wait()

            wait_idx(si1, di1, w1, isem1)
            gd1 = pltpu.async_copy(ht_hbm.at[si1], r1, gsem1)
            scale_scatter(r0, w0, di0)

            @pl.when(c + 2 < cpt)
            def _():
                start_idx(ba + 2 * CHUNK, si0, di0, w0, isem0)

            gd1.wait()

        scale_scatter(r1, w1, di1)
        plsc.subcore_barrier()
        pltpu.sync_copy(acc_sh.at[my_rows], out_hbm.at[cid].at[my_rows])

    return k(ht, src, dst, w, zeros)


def _tc(body, out_shape, *args):
    return pl.pallas_call(body, out_shape=out_shape)(*args)


def _mm_body(x_ref, w_ref, o_ref):
    o_ref[...] = jnp.dot(x_ref[...], w_ref[...],
                         preferred_element_type=jnp.float32)


def _dinv_ht_body(p0_ref, p1_ref, xw_ref, dinv_ref, ht_ref):
    dinv = lax.rsqrt(1.0 + p0_ref[...] + p1_ref[...])
    dinv_ref[...] = dinv
    ht_ref[...] = dinv * xw_ref[...]


def _combine_body(dinv_ref, a0_ref, a1_ref, hlin_ref, w_ref, b_ref,
                  olin_ref, oht_ref):
    dinv = dinv_ref[...]
    h = jnp.maximum(
        dinv * (a0_ref[...] + a1_ref[...])
        + dinv * dinv * hlin_ref[...] + b_ref[...], 0.0)
    hw = jnp.dot(h, w_ref[...], preferred_element_type=jnp.float32)
    olin_ref[...] = hw
    oht_ref[...] = dinv * hw


def _final_body(dinv_ref, a0_ref, a1_ref, hlin_ref, wh_ref, b_ref, bh_ref,
                o_ref):
    dinv = dinv_ref[...]
    h = jnp.maximum(
        dinv * (a0_ref[...] + a1_ref[...])
        + dinv * dinv * hlin_ref[...] + b_ref[...], 0.0)
    o_ref[...] = jnp.dot(h, wh_ref[...],
                         preferred_element_type=jnp.float32) + bh_ref[...]


@jax.jit
def kernel(x, edge_index, edge_weight, W1, b1, W2, b2, Wh, bh):
    n = x.shape[0]
    e = edge_weight.shape[0]
    npad = ((n + NS * 8 - 1) // (NS * 8)) * (NS * 8)
    egrain = NC * NS * CHUNK * 2
    ep = ((e + egrain - 1) // egrain) * egrain

    src = edge_index[0].astype(jnp.int32)
    dst = edge_index[1].astype(jnp.int32)
    w = edge_weight.astype(jnp.float32)
    if ep != e:
        pad = ep - e
        src = jnp.concatenate([src, jnp.zeros((pad,), jnp.int32)])
        dst = jnp.concatenate([dst, jnp.zeros((pad,), jnp.int32)])
        w = jnp.concatenate([w, jnp.zeros((pad,), jnp.float32)])

    f32 = jnp.float32
    dh = W1.shape[1]

    degp = _sc_degree(dst, w, npad)
    xw1 = _tc(_mm_body, jax.ShapeDtypeStruct((n, dh), f32), x, W1)

    p0 = degp[0, :n, 0:1]
    p1 = degp[1, :n, 0:1]
    dinv, h1t = _tc(
        _dinv_ht_body,
        (jax.ShapeDtypeStruct((n, 1), f32),
         jax.ShapeDtypeStruct((n, dh), f32)),
        p0, p1, xw1)

    agg1 = _sc_aggregate(h1t, src, dst, w, npad)
    h1w2, h2t = _tc(
        _combine_body,
        (jax.ShapeDtypeStruct((n, dh), f32),
         jax.ShapeDtypeStruct((n, dh), f32)),
        dinv, agg1[0, :n, :], agg1[1, :n, :], xw1, W2, b1.reshape(1, dh))

    agg2 = _sc_aggregate(h2t, src, dst, w, npad)
    scores = _tc(
        _final_body,
        jax.ShapeDtypeStruct((n, 1), f32),
        dinv, agg2[0, :n, :], agg2[1, :n, :], h1w2, Wh,
        b2.reshape(1, dh), bh.reshape(1, 1))

    return jnp.squeeze(scores, -1)


# speedup vs baseline: 13.2942x; 1.0669x over previous
import dataclasses
import functools

import jax
import jax.numpy as jnp
from jax import lax
from jax.experimental import pallas as pl
from jax.experimental.pallas import tpu as pltpu
from jax.experimental.pallas import tpu_sc as plsc

NC = 2
NS = 16
LANES = 16
CHUNK = 128


def _sc_mesh():
    return plsc.VectorSubcoreMesh(core_axis_name="c", subcore_axis_name="s")


def _sc_params():
    cp = pltpu.CompilerParams()
    if "needs_layout_passes" in pltpu.CompilerParams.__dataclass_fields__:
        cp = dataclasses.replace(cp, needs_layout_passes=False)
    cp = dataclasses.replace(cp, use_tc_tiling_on_sc=False)
    return cp


def _sc_degree(dst, w, npad):
    ep = dst.shape[0]
    cpt = ep // (NC * NS * CHUNK)

    @functools.partial(
        pl.kernel,
        out_type=jax.ShapeDtypeStruct((NC, NS, npad), jnp.float32),
        mesh=_sc_mesh(),
        scratch_types=[
            pltpu.VMEM((npad,), jnp.float32),
            pltpu.VMEM((CHUNK,), jnp.int32),
            pltpu.VMEM((CHUNK,), jnp.int32),
            pltpu.VMEM((CHUNK,), jnp.float32),
            pltpu.VMEM((CHUNK,), jnp.float32),
            pltpu.SemaphoreType.DMA,
            pltpu.SemaphoreType.DMA,
        ],
        compiler_params=_sc_params(),
    )
    def k(dst_hbm, w_hbm, out_hbm, deg_v, di0, di1, w0, w1, sem0, sem1):
        cid = lax.axis_index("c")
        sid = lax.axis_index("s")
        wid = cid * NS + sid
        base0 = wid * cpt * CHUNK

        def start_idx(base, di, wb, sem):
            pltpu.async_copy(dst_hbm.at[pl.ds(base, CHUNK)], di, sem)
            pltpu.async_copy(w_hbm.at[pl.ds(base, CHUNK)], wb, sem)

        def wait_idx(di, wb, sem):
            pltpu.make_async_copy(dst_hbm.at[pl.ds(0, CHUNK)], di, sem).wait()
            pltpu.make_async_copy(w_hbm.at[pl.ds(0, CHUNK)], wb, sem).wait()

        def accum(di, wb):
            @pl.loop(0, CHUNK, step=LANES, unroll=4)
            def _(j):
                sl = pl.ds(j, LANES)
                plsc.addupdate_scatter(deg_v, [di[sl]], wb[sl])

        start_idx(base0, di0, w0, sem0)

        zero16 = jnp.zeros((LANES,), jnp.float32)

        @pl.loop(0, npad, step=LANES, unroll=8)
        def _(i):
            deg_v[pl.ds(i, LANES)] = zero16

        @pl.loop(0, cpt, step=2)
        def _(c):
            ba = base0 + c * CHUNK
            wait_idx(di0, w0, sem0)
            start_idx(ba + CHUNK, di1, w1, sem1)
            accum(di0, w0)
            wait_idx(di1, w1, sem1)

            @pl.when(c + 2 < cpt)
            def _():
                start_idx(ba + 2 * CHUNK, di0, w0, sem0)

            accum(di1, w1)

        pltpu.sync_copy(deg_v, out_hbm.at[cid].at[sid])

    return k(dst, w)


def _sc_aggregate(ht, src, dst, w, npad):
    ep = src.shape[0]
    d = ht.shape[1]
    cpt = ep // (NC * NS * CHUNK)
    rows_per_tile = npad // NS
    zeros = jnp.zeros((rows_per_tile, d), jnp.float32)

    @functools.partial(
        pl.kernel,
        out_type=jax.ShapeDtypeStruct((NC, npad, d), jnp.float32),
        mesh=_sc_mesh(),
        scratch_types=[
            pltpu.VMEM((CHUNK,), jnp.int32),
            pltpu.VMEM((CHUNK,), jnp.int32),
            pltpu.VMEM((CHUNK,), jnp.int32),
            pltpu.VMEM((CHUNK,), jnp.int32),
            pltpu.VMEM((CHUNK,), jnp.float32),
            pltpu.VMEM((CHUNK,), jnp.float32),
            pltpu.VMEM((CHUNK, d), jnp.float32),
            pltpu.VMEM((CHUNK, d), jnp.float32),
            pltpu.VMEM_SHARED((npad, d), jnp.float32),
            pltpu.SemaphoreType.DMA,
            pltpu.SemaphoreType.DMA,
            pltpu.SemaphoreType.DMA,
            pltpu.SemaphoreType.DMA,
        ],
        compiler_params=_sc_params(),
    )
    def k(ht_hbm, src_hbm, dst_hbm, w_hbm, zero_hbm, out_hbm,
          si0, si1, di0, di1, w0, w1, r0, r1, acc_sh,
          isem0, isem1, gsem0, gsem1):
        cid = lax.axis_index("c")
        sid = lax.axis_index("s")
        wid = cid * NS + sid
        my_rows = pl.ds(sid * rows_per_tile, rows_per_tile)
        base0 = wid * cpt * CHUNK

        def start_idx(base, si, di, wb, sem):
            pltpu.async_copy(src_hbm.at[pl.ds(base, CHUNK)], si, sem)
            pltpu.async_copy(dst_hbm.at[pl.ds(base, CHUNK)], di, sem)
            pltpu.async_copy(w_hbm.at[pl.ds(base, CHUNK)], wb, sem)

        def wait_idx(si, di, wb, sem):
            pltpu.make_async_copy(src_hbm.at[pl.ds(0, CHUNK)], si, sem).wait()
            pltpu.make_async_copy(dst_hbm.at[pl.ds(0, CHUNK)], di, sem).wait()
            pltpu.make_async_copy(w_hbm.at[pl.ds(0, CHUNK)], wb, sem).wait()

        def scale_scatter(rows, wb, di):
            @pl.loop(0, CHUNK)
            def _(j):
                sp = jnp.full((LANES,), j, jnp.int32)
                wj = plsc.load_gather(wb, [sp])
                for kk in range(d // LANES):
                    sl = pl.ds(kk * LANES, LANES)
                    rows[j, sl] = rows[j, sl] * wj

            pltpu.sync_copy(rows, acc_sh.at[di], add=True)

        start_idx(base0, si0, di0, w0, isem0)
        pltpu.sync_copy(zero_hbm, acc_sh.at[my_rows])
        plsc.subcore_barrier()

        @pl.loop(0, cpt, step=2)
        def _(c):
            ba = base0 + c * CHUNK
            wait_idx(si0, di0, w0, isem0)
            gd0 = pltpu.async_copy(ht_hbm.at[si0], r0, gsem0)

            @pl.when(c > 0)
            def _():
                scale_scatter(r1, w1, di1)

            start_idx(ba + CHUNK, si1, di1, w1, isem1)
            gd0.wait()

            wait_idx(si1, di1, w1, isem1)
            gd1 = pltpu.async_copy(ht_hbm.at[si1], r1, gsem1)
            scale_scatter(r0, w0, di0)

            @pl.when(c + 2 < cpt)
            def _():
                start_idx(ba + 2 * CHUNK, si0, di0, w0, isem0)

            gd1.wait()

        scale_scatter(r1, w1, di1)
        plsc.subcore_barrier()
        pltpu.sync_copy(acc_sh.at[my_rows], out_hbm.at[cid].at[my_rows])

    return k(ht, src, dst, w, zeros)


def _tc(body, out_shape, *args):
    return pl.pallas_call(body, out_shape=out_shape)(*args)


def _mm_body(x_ref, w_ref, o_ref):
    o_ref[...] = jnp.dot(x_ref[...], w_ref[...],
                         preferred_element_type=jnp.float32)


def _dinv_ht_body(pt_ref, xw_ref, dinv_ref, ht_ref):
    deg = 1.0 + jnp.sum(pt_ref[...], axis=1, keepdims=True)
    dinv = lax.rsqrt(deg)
    dinv_ref[...] = dinv
    ht_ref[...] = dinv * xw_ref[...]


def _combine_body(dinv_ref, a0_ref, a1_ref, hlin_ref, w_ref, b_ref,
                  olin_ref, oht_ref):
    dinv = dinv_ref[...]
    h = jnp.maximum(
        dinv * (a0_ref[...] + a1_ref[...])
        + dinv * dinv * hlin_ref[...] + b_ref[...], 0.0)
    hw = jnp.dot(h, w_ref[...], preferred_element_type=jnp.float32)
    olin_ref[...] = hw
    oht_ref[...] = dinv * hw


def _final_body(dinv_ref, a0_ref, a1_ref, hlin_ref, wh_ref, b_ref, bh_ref,
                o_ref):
    dinv = dinv_ref[...]
    h = jnp.maximum(
        dinv * (a0_ref[...] + a1_ref[...])
        + dinv * dinv * hlin_ref[...] + b_ref[...], 0.0)
    o_ref[...] = jnp.dot(h, wh_ref[...],
                         preferred_element_type=jnp.float32) + bh_ref[...]


@jax.jit
def kernel(x, edge_index, edge_weight, W1, b1, W2, b2, Wh, bh):
    n = x.shape[0]
    e = edge_weight.shape[0]
    npad = ((n + NS * 8 - 1) // (NS * 8)) * (NS * 8)
    egrain = NC * NS * CHUNK * 2
    ep = ((e + egrain - 1) // egrain) * egrain

    src = edge_index[0].astype(jnp.int32)
    dst = edge_index[1].astype(jnp.int32)
    w = edge_weight.astype(jnp.float32)
    if ep != e:
        pad = ep - e
        src = jnp.concatenate([src, jnp.zeros((pad,), jnp.int32)])
        dst = jnp.concatenate([dst, jnp.zeros((pad,), jnp.int32)])
        w = jnp.concatenate([w, jnp.zeros((pad,), jnp.float32)])

    f32 = jnp.float32
    dh = W1.shape[1]

    degp = _sc_degree(dst, w, npad)
    xw1 = _tc(_mm_body, jax.ShapeDtypeStruct((n, dh), f32), x, W1)

    pt = degp.reshape(NC * NS, npad).T[:n, :]
    dinv, h1t = _tc(
        _dinv_ht_body,
        (jax.ShapeDtypeStruct((n, 1), f32),
         jax.ShapeDtypeStruct((n, dh), f32)),
        pt, xw1)

    agg1 = _sc_aggregate(h1t, src, dst, w, npad)
    h1w2, h2t = _tc(
        _combine_body,
        (jax.ShapeDtypeStruct((n, dh), f32),
         jax.ShapeDtypeStruct((n, dh), f32)),
        dinv, agg1[0, :n, :], agg1[1, :n, :], xw1, W2, b1.reshape(1, dh))

    agg2 = _sc_aggregate(h2t, src, dst, w, npad)
    scores = _tc(
        _final_body,
        jax.ShapeDtypeStruct((n, 1), f32),
        dinv, agg2[0, :n, :], agg2[1, :n, :], h1w2, Wh,
        b2.reshape(1, dh), bh.reshape(1, 1))

    return jnp.squeeze(scores, -1)


# speedup vs baseline: 13.3126x; 1.0014x over previous
import dataclasses
import functools

import jax
import jax.numpy as jnp
from jax import lax
from jax.experimental import pallas as pl
from jax.experimental.pallas import tpu as pltpu
from jax.experimental.pallas import tpu_sc as plsc

NC = 2
NS = 16
LANES = 16
CHUNK = 128


def _sc_mesh():
    return plsc.VectorSubcoreMesh(core_axis_name="c", subcore_axis_name="s")


def _sc_params():
    cp = pltpu.CompilerParams()
    if "needs_layout_passes" in pltpu.CompilerParams.__dataclass_fields__:
        cp = dataclasses.replace(cp, needs_layout_passes=False)
    cp = dataclasses.replace(cp, use_tc_tiling_on_sc=False)
    return cp


def _sc_degree(dst, w, npad):
    ep = dst.shape[0]
    cpt = ep // (NC * NS * CHUNK)

    @functools.partial(
        pl.kernel,
        out_type=jax.ShapeDtypeStruct((NC, NS, npad), jnp.float32),
        mesh=_sc_mesh(),
        scratch_types=[
            pltpu.VMEM((npad,), jnp.float32),
            pltpu.VMEM((CHUNK,), jnp.int32),
            pltpu.VMEM((CHUNK,), jnp.int32),
            pltpu.VMEM((CHUNK,), jnp.float32),
            pltpu.VMEM((CHUNK,), jnp.float32),
            pltpu.SemaphoreType.DMA,
            pltpu.SemaphoreType.DMA,
        ],
        compiler_params=_sc_params(),
    )
    def k(dst_hbm, w_hbm, out_hbm, deg_v, di0, di1, w0, w1, sem0, sem1):
        cid = lax.axis_index("c")
        sid = lax.axis_index("s")
        wid = cid * NS + sid
        base0 = wid * cpt * CHUNK

        def start_idx(base, di, wb, sem):
            pltpu.async_copy(dst_hbm.at[pl.ds(base, CHUNK)], di, sem)
            pltpu.async_copy(w_hbm.at[pl.ds(base, CHUNK)], wb, sem)

        def wait_idx(di, wb, sem):
            pltpu.make_async_copy(dst_hbm.at[pl.ds(0, CHUNK)], di, sem).wait()
            pltpu.make_async_copy(w_hbm.at[pl.ds(0, CHUNK)], wb, sem).wait()

        def accum(di, wb):
            @pl.loop(0, CHUNK, step=LANES, unroll=4)
            def _(j):
                sl = pl.ds(j, LANES)
                plsc.addupdate_scatter(deg_v, [di[sl]], wb[sl])

        start_idx(base0, di0, w0, sem0)

        zero16 = jnp.zeros((LANES,), jnp.float32)

        @pl.loop(0, npad, step=LANES, unroll=8)
        def _(i):
            deg_v[pl.ds(i, LANES)] = zero16

        @pl.loop(0, cpt, step=2)
        def _(c):
            ba = base0 + c * CHUNK
            wait_idx(di0, w0, sem0)
            start_idx(ba + CHUNK, di1, w1, sem1)
            accum(di0, w0)
            wait_idx(di1, w1, sem1)

            @pl.when(c + 2 < cpt)
            def _():
                start_idx(ba + 2 * CHUNK, di0, w0, sem0)

            accum(di1, w1)

        pltpu.sync_copy(deg_v, out_hbm.at[cid].at[sid])

    return k(dst, w)


def _sc_aggregate(ht, src, dst, w, npad):
    ep = src.shape[0]
    d = ht.shape[1]
    cpt = ep // (NC * NS * CHUNK)
    rows_per_tile = npad // NS
    zeros = jnp.zeros((rows_per_tile, d), jnp.float32)

    @functools.partial(
        pl.kernel,
        out_type=jax.ShapeDtypeStruct((NC, npad, d), jnp.float32),
        mesh=_sc_mesh(),
        scratch_types=[
            pltpu.VMEM((CHUNK,), jnp.int32),
            pltpu.VMEM((CHUNK,), jnp.int32),
            pltpu.VMEM((CHUNK,), jnp.int32),
            pltpu.VMEM((CHUNK,), jnp.int32),
            pltpu.VMEM((CHUNK,), jnp.float32),
            pltpu.VMEM((CHUNK,), jnp.float32),
            pltpu.VMEM((CHUNK, d), jnp.float32),
            pltpu.VMEM((CHUNK, d), jnp.float32),
            pltpu.VMEM_SHARED((npad, d), jnp.float32),
            pltpu.SemaphoreType.DMA,
            pltpu.SemaphoreType.DMA,
            pltpu.SemaphoreType.DMA,
            pltpu.SemaphoreType.DMA,
        ],
        compiler_params=_sc_params(),
    )
    def k(ht_hbm, src_hbm, dst_hbm, w_hbm, zero_hbm, out_hbm,
          si0, si1, di0, di1, w0, w1, r0, r1, acc_sh,
          isem0, isem1, gsem0, gsem1):
        cid = lax.axis_index("c")
        sid = lax.axis_index("s")
        wid = cid * NS + sid
        my_rows = pl.ds(sid * rows_per_tile, rows_per_tile)
        base0 = wid * cpt * CHUNK

        def start_idx(base, si, di, wb, sem):
            pltpu.async_copy(src_hbm.at[pl.ds(base, CHUNK)], si, sem)
            pltpu.async_copy(dst_hbm.at[pl.ds(base, CHUNK)], di, sem)
            pltpu.async_copy(w_hbm.at[pl.ds(base, CHUNK)], wb, sem)

        def wait_idx(si, di, wb, sem):
            pltpu.make_async_copy(src_hbm.at[pl.ds(0, CHUNK)], si, sem).wait()
            pltpu.make_async_copy(dst_hbm.at[pl.ds(0, CHUNK)], di, sem).wait()
            pltpu.make_async_copy(w_hbm.at[pl.ds(0, CHUNK)], wb, sem).wait()

        def scale_scatter(rows, wb, di):
            @pl.loop(0, CHUNK, step=LANES)
            def _(g):
                gs = jnp.full((LANES,), g, jnp.int32)
                for l in range(LANES):
                    wj = plsc.load_gather(wb, [gs + l])
                    for kk in range(d // LANES):
                        sl = pl.ds(kk * LANES, LANES)
                        rows[g + l, sl] = rows[g + l, sl] * wj

            pltpu.sync_copy(rows, acc_sh.at[di], add=True)

        start_idx(base0, si0, di0, w0, isem0)
        pltpu.sync_copy(zero_hbm, acc_sh.at[my_rows])
        plsc.subcore_barrier()

        @pl.loop(0, cpt, step=2)
        def _(c):
            ba = base0 + c * CHUNK
            wait_idx(si0, di0, w0, isem0)
            gd0 = pltpu.async_copy(ht_hbm.at[si0], r0, gsem0)

            @pl.when(c > 0)
            def _():
                scale_scatter(r1, w1, di1)

            start_idx(ba + CHUNK, si1, di1, w1, isem1)
            gd0.wait()

            wait_idx(si1, di1, w1, isem1)
            gd1 = pltpu.async_copy(ht_hbm.at[si1], r1, gsem1)
            scale_scatter(r0, w0, di0)

            @pl.when(c + 2 < cpt)
            def _():
                start_idx(ba + 2 * CHUNK, si0, di0, w0, isem0)

            gd1.wait()

        scale_scatter(r1, w1, di1)
        plsc.subcore_barrier()
        pltpu.sync_copy(acc_sh.at[my_rows], out_hbm.at[cid].at[my_rows])

    return k(ht, src, dst, w, zeros)


def _tc(body, out_shape, *args):
    return pl.pallas_call(body, out_shape=out_shape)(*args)


def _mm_body(x_ref, w_ref, o_ref):
    o_ref[...] = jnp.dot(x_ref[...], w_ref[...],
                         preferred_element_type=jnp.float32)


def _dinv_ht_body(pt_ref, xw_ref, dinv_ref, ht_ref):
    deg = 1.0 + jnp.sum(pt_ref[...], axis=1, keepdims=True)
    dinv = lax.rsqrt(deg)
    dinv_ref[...] = dinv
    ht_ref[...] = dinv * xw_ref[...]


def _combine_body(dinv_ref, a0_ref, a1_ref, hlin_ref, w_ref, b_ref,
                  olin_ref, oht_ref):
    dinv = dinv_ref[...]
    h = jnp.maximum(
        dinv * (a0_ref[...] + a1_ref[...])
        + dinv * dinv * hlin_ref[...] + b_ref[...], 0.0)
    hw = jnp.dot(h, w_ref[...], preferred_element_type=jnp.float32)
    olin_ref[...] = hw
    oht_ref[...] = dinv * hw


def _final_body(dinv_ref, a0_ref, a1_ref, hlin_ref, wh_ref, b_ref, bh_ref,
                o_ref):
    dinv = dinv_ref[...]
    h = jnp.maximum(
        dinv * (a0_ref[...] + a1_ref[...])
        + dinv * dinv * hlin_ref[...] + b_ref[...], 0.0)
    o_ref[...] = jnp.dot(h, wh_ref[...],
                         preferred_element_type=jnp.float32) + bh_ref[...]


@jax.jit
def kernel(x, edge_index, edge_weight, W1, b1, W2, b2, Wh, bh):
    n = x.shape[0]
    e = edge_weight.shape[0]
    npad = ((n + NS * 8 - 1) // (NS * 8)) * (NS * 8)
    egrain = NC * NS * CHUNK * 2
    ep = ((e + egrain - 1) // egrain) * egrain

    src = edge_index[0].astype(jnp.int32)
    dst = edge_index[1].astype(jnp.int32)
    w = edge_weight.astype(jnp.float32)
    if ep != e:
        pad = ep - e
        src = jnp.concatenate([src, jnp.zeros((pad,), jnp.int32)])
        dst = jnp.concatenate([dst, jnp.zeros((pad,), jnp.int32)])
        w = jnp.concatenate([w, jnp.zeros((pad,), jnp.float32)])

    f32 = jnp.float32
    dh = W1.shape[1]

    degp = _sc_degree(dst, w, npad)
    xw1 = _tc(_mm_body, jax.ShapeDtypeStruct((n, dh), f32), x, W1)

    pt = degp.reshape(NC * NS, npad).T[:n, :]
    dinv, h1t = _tc(
        _dinv_ht_body,
        (jax.ShapeDtypeStruct((n, 1), f32),
         jax.ShapeDtypeStruct((n, dh), f32)),
        pt, xw1)

    agg1 = _sc_aggregate(h1t, src, dst, w, npad)
    h1w2, h2t = _tc(
        _combine_body,
        (jax.ShapeDtypeStruct((n, dh), f32),
         jax.ShapeDtypeStruct((n, dh), f32)),
        dinv, agg1[0, :n, :], agg1[1, :n, :], xw1, W2, b1.reshape(1, dh))

    agg2 = _sc_aggregate(h2t, src, dst, w, npad)
    scores = _tc(
        _final_body,
        jax.ShapeDtypeStruct((n, 1), f32),
        dinv, agg2[0, :n, :], agg2[1, :n, :], h1w2, Wh,
        b2.reshape(1, dh), bh.reshape(1, 1))

    return jnp.squeeze(scores, -1)


# speedup vs baseline: 14.5762x; 1.0949x over previous
import dataclasses
import functools

import jax
import jax.numpy as jnp
from jax import lax
from jax.experimental import pallas as pl
from jax.experimental.pallas import tpu as pltpu
from jax.experimental.pallas import tpu_sc as plsc

NC = 2
NS = 16
LANES = 16
CHUNK = 128


def _sc_mesh():
    return plsc.VectorSubcoreMesh(core_axis_name="c", subcore_axis_name="s")


def _sc_params():
    cp = pltpu.CompilerParams()
    if "needs_layout_passes" in pltpu.CompilerParams.__dataclass_fields__:
        cp = dataclasses.replace(cp, needs_layout_passes=False)
    cp = dataclasses.replace(cp, use_tc_tiling_on_sc=False)
    return cp


def _sc_degree(dst, w, npad):
    ep = dst.shape[0]
    cpt = ep // (NC * NS * CHUNK)

    @functools.partial(
        pl.kernel,
        out_type=jax.ShapeDtypeStruct((NC, NS, npad), jnp.float32),
        mesh=_sc_mesh(),
        scratch_types=[
            pltpu.VMEM((npad,), jnp.float32),
            pltpu.VMEM((CHUNK,), jnp.int32),
            pltpu.VMEM((CHUNK,), jnp.int32),
            pltpu.VMEM((CHUNK,), jnp.float32),
            pltpu.VMEM((CHUNK,), jnp.float32),
            pltpu.SemaphoreType.DMA,
            pltpu.SemaphoreType.DMA,
        ],
        compiler_params=_sc_params(),
    )
    def k(dst_hbm, w_hbm, out_hbm, deg_v, di0, di1, w0, w1, sem0, sem1):
        cid = lax.axis_index("c")
        sid = lax.axis_index("s")
        wid = cid * NS + sid
        base0 = wid * cpt * CHUNK

        def start_idx(base, di, wb, sem):
            pltpu.async_copy(dst_hbm.at[pl.ds(base, CHUNK)], di, sem)
            pltpu.async_copy(w_hbm.at[pl.ds(base, CHUNK)], wb, sem)

        def wait_idx(di, wb, sem):
            pltpu.make_async_copy(dst_hbm.at[pl.ds(0, CHUNK)], di, sem).wait()
            pltpu.make_async_copy(w_hbm.at[pl.ds(0, CHUNK)], wb, sem).wait()

        def accum(di, wb):
            @pl.loop(0, CHUNK, step=LANES, unroll=4)
            def _(j):
                sl = pl.ds(j, LANES)
                plsc.addupdate_scatter(deg_v, [di[sl]], wb[sl])

        start_idx(base0, di0, w0, sem0)

        zero16 = jnp.zeros((LANES,), jnp.float32)

        @pl.loop(0, npad, step=LANES, unroll=8)
        def _(i):
            deg_v[pl.ds(i, LANES)] = zero16

        @pl.loop(0, cpt, step=2)
        def _(c):
            ba = base0 + c * CHUNK
            wait_idx(di0, w0, sem0)
            start_idx(ba + CHUNK, di1, w1, sem1)
            accum(di0, w0)
            wait_idx(di1, w1, sem1)

            @pl.when(c + 2 < cpt)
            def _():
                start_idx(ba + 2 * CHUNK, di0, w0, sem0)

            accum(di1, w1)

        pltpu.sync_copy(deg_v, out_hbm.at[cid].at[sid])

    return k(dst, w)


def _sc_aggregate(ht, src, dst, w, npad):
    ep = src.shape[0]
    d = ht.shape[1]
    cpt = ep // (NC * NS * CHUNK)
    rows_per_tile = npad // NS
    zeros = jnp.zeros((rows_per_tile, d), jnp.float32)

    nb = 4
    nd = 8

    @functools.partial(
        pl.kernel,
        out_type=jax.ShapeDtypeStruct((NC, npad, d), jnp.float32),
        mesh=_sc_mesh(),
        scratch_types=[
            [pltpu.VMEM((CHUNK,), jnp.int32) for _ in range(nb)],
            [pltpu.VMEM((CHUNK,), jnp.int32) for _ in range(nd)],
            [pltpu.VMEM((CHUNK,), jnp.float32) for _ in range(nb)],
            [pltpu.VMEM((CHUNK, d), jnp.float32) for _ in range(nb)],
            pltpu.VMEM_SHARED((npad, d), jnp.float32),
            [pltpu.SemaphoreType.DMA for _ in range(nb)],
            [pltpu.SemaphoreType.DMA for _ in range(nd)],
            [pltpu.SemaphoreType.DMA for _ in range(nb)],
            [pltpu.SemaphoreType.DMA for _ in range(nb)],
        ],
        compiler_params=_sc_params(),
    )
    def k(ht_hbm, src_hbm, dst_hbm, w_hbm, zero_hbm, out_hbm,
          si, di, wv, rv, acc_sh, isem, dsem, gsem, ssem):
        cid = lax.axis_index("c")
        sid = lax.axis_index("s")
        wid = cid * NS + sid
        my_rows = pl.ds(sid * rows_per_tile, rows_per_tile)
        base0 = wid * cpt * CHUNK

        def start_siw(t, q):
            pltpu.async_copy(src_hbm.at[pl.ds(base0 + t * CHUNK, CHUNK)],
                             si[q], isem[q])
            pltpu.async_copy(w_hbm.at[pl.ds(base0 + t * CHUNK, CHUNK)],
                             wv[q], isem[q])

        def start_di(t, m):
            pltpu.async_copy(dst_hbm.at[pl.ds(base0 + t * CHUNK, CHUNK)],
                             di[m], dsem[m])

        def wait_siw(q):
            pltpu.make_async_copy(src_hbm.at[pl.ds(0, CHUNK)], si[q],
                                  isem[q]).wait()
            pltpu.make_async_copy(w_hbm.at[pl.ds(0, CHUNK)], wv[q],
                                  isem[q]).wait()

        def wait_di(m):
            pltpu.make_async_copy(dst_hbm.at[pl.ds(0, CHUNK)], di[m],
                                  dsem[m]).wait()

        def wait_gather(q):
            pltpu.make_async_copy(ht_hbm.at[si[q]], rv[q], gsem[q]).wait()

        def wait_scatter(q, m):
            pltpu.make_async_copy(rv[q], acc_sh.at[di[m]], ssem[q]).wait()

        def scale(q):
            rows = rv[q]
            wb = wv[q]

            @pl.loop(0, CHUNK, step=LANES)
            def _(g):
                gs = jnp.full((LANES,), g, jnp.int32)
                for l in range(LANES):
                    wj = plsc.load_gather(wb, [gs + l])
                    for kk in range(d // LANES):
                        sl = pl.ds(kk * LANES, LANES)
                        rows[g + l, sl] = rows[g + l, sl] * wj

        for q in range(nb):
            start_siw(q, q)
        for m in range(nd):
            start_di(m, m)
        pltpu.sync_copy(zero_hbm, acc_sh.at[my_rows])
        plsc.subcore_barrier()

        @pl.loop(0, cpt, step=nd)
        def _(c):
            for p in range(nd):
                t = c + p
                q = p % nb
                m = p
                prevq = (p - 1) % nb
                prevm = (p - 1) % nd

                @pl.when(t >= nb)
                def _():
                    wait_scatter(q, (p + nb) % nd)

                    @pl.when(t + nb < cpt)
                    def _():
                        start_di(t + nb, (p + nb) % nd)

                wait_siw(q)
                wait_di(m)
                pltpu.async_copy(ht_hbm.at[si[q]], rv[q], gsem[q])

                @pl.when(t >= 1)
                def _():
                    wait_gather(prevq)
                    scale(prevq)
                    pltpu.async_copy(rv[prevq], acc_sh.at[di[prevm]],
                                     ssem[prevq], add=True)

                    @pl.when(t + nb - 1 < cpt)
                    def _():
                        start_siw(t + nb - 1, prevq)

        wait_gather((cpt - 1) % nb)
        scale((cpt - 1) % nb)
        pltpu.async_copy(rv[(cpt - 1) % nb], acc_sh.at[di[(cpt - 1) % nd]],
                         ssem[(cpt - 1) % nb], add=True)
        for q in range(nb):
            wait_scatter(q, q)
        plsc.subcore_barrier()
        pltpu.sync_copy(acc_sh.at[my_rows], out_hbm.at[cid].at[my_rows])

    return k(ht, src, dst, w, zeros)


def _tc(body, out_shape, *args):
    return pl.pallas_call(body, out_shape=out_shape)(*args)


def _mm_body(x_ref, w_ref, o_ref):
    o_ref[...] = jnp.dot(x_ref[...], w_ref[...],
                         preferred_element_type=jnp.float32)


def _dinv_ht_body(pt_ref, xw_ref, dinv_ref, ht_ref):
    deg = 1.0 + jnp.sum(pt_ref[...], axis=1, keepdims=True)
    dinv = lax.rsqrt(deg)
    dinv_ref[...] = dinv
    ht_ref[...] = dinv * xw_ref[...]


def _combine_body(dinv_ref, a0_ref, a1_ref, hlin_ref, w_ref, b_ref,
                  olin_ref, oht_ref):
    dinv = dinv_ref[...]
    h = jnp.maximum(
        dinv * (a0_ref[...] + a1_ref[...])
        + dinv * dinv * hlin_ref[...] + b_ref[...], 0.0)
    hw = jnp.dot(h, w_ref[...], preferred_element_type=jnp.float32)
    olin_ref[...] = hw
    oht_ref[...] = dinv * hw


def _final_body(dinv_ref, a0_ref, a1_ref, hlin_ref, wh_ref, b_ref, bh_ref,
                o_ref):
    dinv = dinv_ref[...]
    h = jnp.maximum(
        dinv * (a0_ref[...] + a1_ref[...])
        + dinv * dinv * hlin_ref[...] + b_ref[...], 0.0)
    o_ref[...] = jnp.dot(h, wh_ref[...],
                         preferred_element_type=jnp.float32) + bh_ref[...]


@jax.jit
def kernel(x, edge_index, edge_weight, W1, b1, W2, b2, Wh, bh):
    n = x.shape[0]
    e = edge_weight.shape[0]
    npad = ((n + NS * 8 - 1) // (NS * 8)) * (NS * 8)
    egrain = NC * NS * CHUNK * 2
    ep = ((e + egrain - 1) // egrain) * egrain

    src = edge_index[0].astype(jnp.int32)
    dst = edge_index[1].astype(jnp.int32)
    w = edge_weight.astype(jnp.float32)
    if ep != e:
        pad = ep - e
        src = jnp.concatenate([src, jnp.zeros((pad,), jnp.int32)])
        dst = jnp.concatenate([dst, jnp.zeros((pad,), jnp.int32)])
        w = jnp.concatenate([w, jnp.zeros((pad,), jnp.float32)])

    f32 = jnp.float32
    dh = W1.shape[1]

    degp = _sc_degree(dst, w, npad)
    xw1 = _tc(_mm_body, jax.ShapeDtypeStruct((n, dh), f32), x, W1)

    pt = degp.reshape(NC * NS, npad).T[:n, :]
    dinv, h1t = _tc(
        _dinv_ht_body,
        (jax.ShapeDtypeStruct((n, 1), f32),
         jax.ShapeDtypeStruct((n, dh), f32)),
        pt, xw1)

    agg1 = _sc_aggregate(h1t, src, dst, w, npad)
    h1w2, h2t = _tc(
        _combine_body,
        (jax.ShapeDtypeStruct((n, dh), f32),
         jax.ShapeDtypeStruct((n, dh), f32)),
        dinv, agg1[0, :n, :], agg1[1, :n, :], xw1, W2, b1.reshape(1, dh))

    agg2 = _sc_aggregate(h2t, src, dst, w, npad)
    scores = _tc(
        _final_body,
        jax.ShapeDtypeStruct((n, 1), f32),
        dinv, agg2[0, :n, :], agg2[1, :n, :], h1w2, Wh,
        b2.reshape(1, dh), bh.reshape(1, 1))

    return jnp.squeeze(scores, -1)
